# Initial kernel scaffold; baseline (speedup 1.0000x reference)
import jax
import jax.numpy as jnp
from jax.experimental import pallas as pl


def kernel(z_what, z_where, z_present, z_depth, W1, b1, W2, b2):
    raise NotImplementedError("write your pallas kernel here")


# speedup vs baseline: 19098.9144x; 19098.9144x over previous
import functools

import jax
import jax.numpy as jnp
from jax.experimental import pallas as pl
from jax.experimental.pallas import tpu as pltpu

B = 4
N_OBJ = 31
N_ALL = N_OBJ + 1
Z_WHAT = 64
IMG = 224
OBJ = 64
HID = 1024
OUTD = 3 * OBJ * OBJ
NB = 1536
HALF = (OBJ - 1) / 2.0


def _decode_body(z_ref, W1_ref, b1_ref, W2_ref, b2_ref, out_ref, h_ref):
    j = pl.program_id(0)

    @pl.when(j == 0)
    def _():
        h = jnp.dot(z_ref[...], W1_ref[...], preferred_element_type=jnp.float32)
        h_ref[...] = jnp.maximum(h + b1_ref[...], 0.0)

    y = jnp.dot(h_ref[...], W2_ref[...], preferred_element_type=jnp.float32)
    out_ref[...] = jax.nn.sigmoid(y + b2_ref[...])


def _interp_rows(lin, ctr, scale, n_src, src_axis, shape):
    s = jnp.maximum(scale, 1e-6)
    coord = (lin - (2.0 * ctr - 1.0)) / s
    p = (coord + 1.0) * HALF
    p0 = jnp.floor(p)
    w = p - p0
    p1 = p0 + 1.0
    src = jax.lax.broadcasted_iota(jnp.int32, shape, src_axis).astype(jnp.float32)
    v0 = (p0 >= 0) & (p0 <= n_src - 1)
    v1 = (p1 >= 0) & (p1 <= n_src - 1)
    t0 = jnp.where((src == p0) & v0, 1.0 - w, 0.0)
    t1 = jnp.where((src == p1) & v1, w, 0.0)
    return t0 + t1


def _merge_body(linr_ref, linc_ref, zw_ref, ld_ref, D_ref, out_ref, acc_ref):
    b = pl.program_id(0)
    i = pl.program_id(1)

    @pl.when(i == 0)
    def _():
        acc_ref[...] = jnp.zeros_like(acc_ref)

    cx = zw_ref[0, 0, 0]
    cy = zw_ref[0, 0, 1]
    sx = zw_ref[0, 0, 2]
    sy = zw_ref[0, 0, 3]

    ry = _interp_rows(linr_ref[...], cy, sy, OBJ, 1, (IMG, OBJ))
    rxt = _interp_rows(linc_ref[...], cx, sx, OBJ, 0, (OBJ, IMG))

    d = D_ref[0]
    a = jnp.dot(d, rxt, preferred_element_type=jnp.float32)

    ld = ld_ref[0]
    m = jnp.max(ld)
    e = jnp.exp(ld - m)
    s = jnp.sum(e)
    lane = jax.lax.broadcasted_iota(jnp.int32, ld.shape, 1)
    w = jnp.sum(jnp.where(lane == i, e, 0.0)) / s

    @pl.when(i < N_OBJ)
    def _():
        for c in range(3):
            mc = jnp.dot(ry, a[c * OBJ:(c + 1) * OBJ, :],
                         preferred_element_type=jnp.float32)
            acc_ref[c] += w * mc

    @pl.when(i == N_OBJ)
    def _():
        for c in range(3):
            mc = jnp.dot(ry, a[c * OBJ:(c + 1) * OBJ, :],
                         preferred_element_type=jnp.float32)
            cur = acc_ref[c]
            out_ref[0, c] = cur + mc * jnp.where(cur < 0.001, 1.0, 0.0)


@jax.jit
def kernel(z_what, z_where, z_present, z_depth, W1, b1, W2, b2):
    zf = z_what.reshape(B * N_ALL, Z_WHAT)

    decoded = pl.pallas_call(
        _decode_body,
        grid=(OUTD // NB,),
        in_specs=[
            pl.BlockSpec((B * N_ALL, Z_WHAT), lambda j: (0, 0)),
            pl.BlockSpec((Z_WHAT, HID), lambda j: (0, 0)),
            pl.BlockSpec((1, HID), lambda j: (0, 0)),
            pl.BlockSpec((HID, NB), lambda j: (0, j)),
            pl.BlockSpec((1, NB), lambda j: (0, j)),
        ],
        out_specs=pl.BlockSpec((B * N_ALL, NB), lambda j: (0, j)),
        out_shape=jax.ShapeDtypeStruct((B * N_ALL, OUTD), jnp.float32),
        scratch_shapes=[pltpu.VMEM((B * N_ALL, HID), jnp.float32)],
    )(zf, W1, b1.reshape(1, HID), W2, b2.reshape(1, OUTD))

    D3 = decoded.reshape(B * N_ALL, 3 * OBJ, OBJ)

    bg_where = jnp.broadcast_to(
        jnp.array([0.5, 0.5, 1.0, 1.0], jnp.float32), (B, 1, 4))
    zw_f = jnp.concatenate([z_where, bg_where], axis=1).reshape(B * N_ALL, 1, 4)

    neg_inf = jnp.full((B, 1), -jnp.inf, jnp.float32)
    dcol = jnp.concatenate([z_depth[..., 0], neg_inf], axis=1)
    pcol = jnp.concatenate([z_present[..., 0], jnp.zeros((B, 1))], axis=1)
    ld = jnp.where(pcol == 1.0, dcol, -jnp.inf).reshape(B, 1, N_ALL)

    lin = jnp.linspace(-1.0, 1.0, IMG).astype(jnp.float32)
    linr = lin.reshape(IMG, 1)
    linc = lin.reshape(1, IMG)

    out = pl.pallas_call(
        _merge_body,
        grid=(B, N_ALL),
        in_specs=[
            pl.BlockSpec((IMG, 1), lambda b, i: (0, 0)),
            pl.BlockSpec((1, IMG), lambda b, i: (0, 0)),
            pl.BlockSpec((1, 1, 4), lambda b, i: (b * N_ALL + i, 0, 0),
                         memory_space=pltpu.SMEM),
            pl.BlockSpec((1, 1, N_ALL), lambda b, i: (b, 0, 0)),
            pl.BlockSpec((1, 3 * OBJ, OBJ), lambda b, i: (b * N_ALL + i, 0, 0)),
        ],
        out_specs=pl.BlockSpec((1, 3, IMG, IMG), lambda b, i: (b, 0, 0, 0)),
        out_shape=jax.ShapeDtypeStruct((B, 3, IMG, IMG), jnp.float32),
        scratch_shapes=[pltpu.VMEM((3, IMG, IMG), jnp.float32)],
    )(linr, linc, zw_f, ld, D3)

    return out
